# baseline (speedup 1.0000x reference)
import functools

import jax
import jax.numpy as jnp
from jax import lax
from jax.experimental import pallas as pl
from jax.experimental.pallas import tpu as pltpu
from jax.experimental.pallas import tpu_sc as plsc

N_REAL = 10000
NPAD = 10048
E_REAL = 160000
EPAD = 163840
EPT = EPAD // 16
SLAB_A = 632
SLAB_B = 624
BR = 1256


def _mm_first(x, wp, nslab):
    n, k = x.shape
    dout = wp.shape[0]
    w = dout // nslab

    def body(x_ref, w_ref, o_ref):
        i = pl.program_id(0)
        y = lax.dot_general(x_ref[...], w_ref[...], (((1,), (1,)), ((), ())),
                            preferred_element_type=jnp.float32)
        row = i * BR + lax.broadcasted_iota(jnp.int32, (BR, dout), 0)
        y = jnp.where(row < n, y, 0.0)
        for q in range(nslab):
            o_ref[q] = y[:, q * w:(q + 1) * w]

    return pl.pallas_call(
        body,
        grid=(NPAD // BR,),
        in_specs=[pl.BlockSpec((BR, k), lambda i: (i, 0)),
                  pl.BlockSpec((dout, k), lambda i: (0, 0))],
        out_specs=pl.BlockSpec((nslab, BR, w), lambda i: (0, i, 0)),
        out_shape=jax.ShapeDtypeStruct((nslab, NPAD, w), jnp.float32),
    )(x, wp)


def _mm_fused(p, b2d, wp, nslab):
    inslab, n, win = p.shape
    din = inslab * win
    dout = wp.shape[0]
    w = dout // nslab

    def body(p_ref, b_ref, w_ref, o_ref):
        i = pl.program_id(0)
        h = jnp.concatenate([p_ref[q] for q in range(inslab)], axis=1)
        h = jnp.maximum(h + b_ref[...], 0.0)
        y = lax.dot_general(h, w_ref[...], (((1,), (1,)), ((), ())),
                            preferred_element_type=jnp.float32)
        row = i * BR + lax.broadcasted_iota(jnp.int32, (BR, dout), 0)
        y = jnp.where(row < N_REAL, y, 0.0)
        for q in range(nslab):
            o_ref[q] = y[:, q * w:(q + 1) * w]

    return pl.pallas_call(
        body,
        grid=(n // BR,),
        in_specs=[pl.BlockSpec((inslab, BR, win), lambda i: (0, i, 0)),
                  pl.BlockSpec((1, din), lambda i: (0, 0)),
                  pl.BlockSpec((dout, din), lambda i: (0, 0))],
        out_specs=pl.BlockSpec((nslab, BR, w), lambda i: (0, i, 0)),
        out_shape=jax.ShapeDtypeStruct((nslab, n, w), jnp.float32),
    )(p, b2d, wp)


def _final_act(p, b2d):
    inslab, n, win = p.shape
    din = inslab * win

    def body(p_ref, b_ref, o_ref):
        h = jnp.concatenate([p_ref[q] for q in range(inslab)], axis=1)
        o_ref[...] = jnp.maximum(h + b_ref[...], 0.0)

    return pl.pallas_call(
        body,
        grid=(n // BR,),
        in_specs=[pl.BlockSpec((inslab, BR, win), lambda i: (0, i, 0)),
                  pl.BlockSpec((1, din), lambda i: (0, 0))],
        out_specs=pl.BlockSpec((BR, din), lambda i: (i, 0)),
        out_shape=jax.ShapeDtypeStruct((n, din), jnp.float32),
    )(p, b2d)


def _slab_copy(src_ref, dst_ref, s):
    @pl.when(s < 8)
    def _():
        b = pl.multiple_of(s * SLAB_A, 8)
        pltpu.sync_copy(src_ref.at[pl.ds(b, SLAB_A)],
                        dst_ref.at[pl.ds(b, SLAB_A)])

    @pl.when(s >= 8)
    def _():
        b = pl.multiple_of(8 * SLAB_A + (s - 8) * SLAB_B, 8)
        pltpu.sync_copy(src_ref.at[pl.ds(b, SLAB_B)],
                        dst_ref.at[pl.ds(b, SLAB_B)])


def _seg_sum(y, src4, dst4, zeros, w, csz, nslab=2, stage=False):
    mesh = plsc.VectorSubcoreMesh(core_axis_name="c", subcore_axis_name="s")

    nbuf = 4
    chunks = EPT // csz
    spc = nslab // 2

    @functools.partial(
        pl.kernel,
        out_type=jax.ShapeDtypeStruct((nslab, NPAD, w), jnp.float32),
        mesh=mesh,
        scratch_types=(
            [pltpu.VMEM((chunks, 1, csz), jnp.int32),
             pltpu.VMEM((chunks, 1, csz), jnp.int32)]
            + [pltpu.VMEM((csz, w), jnp.float32)] * nbuf
            + [pltpu.VMEM_SHARED((NPAD, w), jnp.float32)]
            + ([pltpu.VMEM_SHARED((NPAD, w), jnp.float32)] if stage else [])
            + [pltpu.SemaphoreType.DMA] * (2 * nbuf)
        ),
        compiler_params=pltpu.CompilerParams(use_tc_tiling_on_sc=False),
    )
    def k(y_hbm, src_hbm, dst_hbm, z_hbm, out, *rest):
        if stage:
            (src_v, dst_v, r0, r1, r2, r3, acc_sh, y_sh,
             g0, g1, g2, g3, s0, s1, s2, s3) = rest
        else:
            (src_v, dst_v, r0, r1, r2, r3, acc_sh,
             g0, g1, g2, g3, s0, s1, s2, s3) = rest
            y_sh = None
        bufs = (r0, r1, r2, r3)
        gsem = (g0, g1, g2, g3)
        ssem = (s0, s1, s2, s3)
        c = lax.axis_index("c")
        s = lax.axis_index("s")
        pltpu.sync_copy(src_hbm.at[s], src_v)
        pltpu.sync_copy(dst_hbm.at[s], dst_v)

        def one_pass(sl):
            _slab_copy(z_hbm, acc_sh, s)
            if stage:
                _slab_copy(y_hbm.at[sl], y_sh, s)
            plsc.subcore_barrier()

            def ysrc(b):
                if stage and b >= 2:
                    return y_sh
                return y_hbm.at[sl]

            def gather_start(j, b):
                pltpu.async_copy(ysrc(b).at[src_v.at[j, 0]], bufs[b],
                                 gsem[b])

            def gather_wait(b):
                pltpu.make_async_copy(ysrc(b).at[src_v.at[0, 0]], bufs[b],
                                      gsem[b]).wait()

            def scatter_start(j, b):
                pltpu.async_copy(bufs[b], acc_sh.at[dst_v.at[j, 0]],
                                 ssem[b], add=True)

            def scatter_wait(b):
                pltpu.make_async_copy(bufs[b], acc_sh.at[dst_v.at[0, 0]],
                                      ssem[b]).wait()

            for b in range(nbuf):
                gather_start(b, b)

            def body(u, carry):
                j = 4 * u
                for b in range(nbuf):
                    gather_wait(b)
                    scatter_start(j + b, b)
                for b in range(nbuf):
                    @pl.when(j + nbuf + b < chunks)
                    def _(b=b):
                        scatter_wait(b)
                        gather_start(j + nbuf + b, b)
                return carry

            lax.fori_loop(0, chunks // 4, body, 0)
            for b in range(nbuf):
                scatter_wait(b)
            plsc.subcore_barrier()
            _slab_copy(acc_sh, out.at[sl], s)

        for q in range(spc):
            one_pass(spc * c + q if spc > 1 else c)

    return k(y, src4, dst4, zeros)


def _pad_w(m, r, c):
    return jnp.pad(m.astype(jnp.float32),
                   ((0, r - m.shape[0]), (0, c - m.shape[1])))


def kernel(features, edge_index, W1, b1, W2, b2, W3, b3):
    f32 = jnp.float32
    d1, d2, d3 = 256, 128, 64

    src = edge_index[0].astype(jnp.int32)
    dst = edge_index[1].astype(jnp.int32)
    pad = jnp.full((EPAD - E_REAL,), N_REAL, jnp.int32)
    srcp = jnp.concatenate([src, pad])
    dstp = jnp.concatenate([dst, pad])
    src80 = srcp.reshape(16, EPT // 80, 1, 80)
    dst80 = dstp.reshape(16, EPT // 80, 1, 80)
    src128 = srcp.reshape(16, EPT // 128, 1, 128)
    dst128 = dstp.reshape(16, EPT // 128, 1, 128)
    w1p = _pad_w(W1, d1, features.shape[1])
    w2p = _pad_w(W2, d2, d1)
    w3p = _pad_w(W3, d3, d2)
    b1p = jnp.pad(b1.astype(f32), (0, d1 - b1.shape[0])).reshape(1, d1)
    b2p = jnp.pad(b2.astype(f32), (0, d2 - b2.shape[0])).reshape(1, d2)
    b3p = jnp.pad(b3.astype(f32), (0, d3 - b3.shape[0])).reshape(1, d3)

    zeros64 = jnp.zeros((NPAD, 64), f32)
    y1 = _mm_first(features.astype(f32), w1p, 4)
    p = _seg_sum(y1, src80, dst80, zeros64, 64, 80, nslab=4, stage=True)
    y2 = _mm_fused(p, b1p, w2p, 2)
    p = _seg_sum(y2, src80, dst80, zeros64, 64, 80, nslab=2, stage=True)
    y3 = _mm_fused(p, b2p, w3p, 2)
    p = _seg_sum(y3, src128, dst128, jnp.zeros((NPAD, 32), f32),
                 32, 128, nslab=2, stage=True)
    out = _final_act(p, b3p)
    return out[:N_REAL, :W3.shape[0]]


# speedup vs baseline: 1.1437x; 1.1437x over previous
import functools

import jax
import jax.numpy as jnp
from jax import lax
from jax.experimental import pallas as pl
from jax.experimental.pallas import tpu as pltpu
from jax.experimental.pallas import tpu_sc as plsc

N_REAL = 10000
NPAD = 10048
E_REAL = 160000
EPAD = 163840
EPT = EPAD // 16
SLAB_A = 632
SLAB_B = 624
BR = 1256


def _mm_first(x, wp, nslab):
    n, k = x.shape
    dout = wp.shape[0]
    w = dout // nslab

    def body(x_ref, w_ref, o_ref):
        i = pl.program_id(0)
        y = lax.dot_general(x_ref[...], w_ref[...], (((1,), (1,)), ((), ())),
                            preferred_element_type=jnp.float32)
        row = i * BR + lax.broadcasted_iota(jnp.int32, (BR, dout), 0)
        y = jnp.where(row < n, y, 0.0)
        for q in range(nslab):
            o_ref[q] = y[:, q * w:(q + 1) * w]

    return pl.pallas_call(
        body,
        grid=(NPAD // BR,),
        in_specs=[pl.BlockSpec((BR, k), lambda i: (i, 0)),
                  pl.BlockSpec((dout, k), lambda i: (0, 0))],
        out_specs=pl.BlockSpec((nslab, BR, w), lambda i: (0, i, 0)),
        out_shape=jax.ShapeDtypeStruct((nslab, NPAD, w), jnp.float32),
    )(x, wp)


def _mm_fused(p, b2d, wp, nslab):
    inslab, n, win = p.shape
    din = inslab * win
    dout = wp.shape[0]
    w = dout // nslab

    def body(p_ref, b_ref, w_ref, o_ref):
        i = pl.program_id(0)
        h = jnp.concatenate([p_ref[q] for q in range(inslab)], axis=1)
        h = jnp.maximum(h + b_ref[...], 0.0)
        y = lax.dot_general(h, w_ref[...], (((1,), (1,)), ((), ())),
                            preferred_element_type=jnp.float32)
        row = i * BR + lax.broadcasted_iota(jnp.int32, (BR, dout), 0)
        y = jnp.where(row < N_REAL, y, 0.0)
        for q in range(nslab):
            o_ref[q] = y[:, q * w:(q + 1) * w]

    return pl.pallas_call(
        body,
        grid=(n // BR,),
        in_specs=[pl.BlockSpec((inslab, BR, win), lambda i: (0, i, 0)),
                  pl.BlockSpec((1, din), lambda i: (0, 0)),
                  pl.BlockSpec((dout, din), lambda i: (0, 0))],
        out_specs=pl.BlockSpec((nslab, BR, w), lambda i: (0, i, 0)),
        out_shape=jax.ShapeDtypeStruct((nslab, n, w), jnp.float32),
    )(p, b2d, wp)


def _final_act(p, b2d, dout):
    inslab, n, win = p.shape
    din = inslab * win

    def body(p_ref, b_ref, o_ref):
        h = jnp.concatenate([p_ref[q] for q in range(inslab)], axis=1)
        h = jnp.maximum(h + b_ref[...], 0.0)
        o_ref[...] = h[:, :dout]

    return pl.pallas_call(
        body,
        grid=(n // BR,),
        in_specs=[pl.BlockSpec((inslab, BR, win), lambda i: (0, i, 0)),
                  pl.BlockSpec((1, din), lambda i: (0, 0))],
        out_specs=pl.BlockSpec((BR, dout), lambda i: (i, 0)),
        out_shape=jax.ShapeDtypeStruct((N_REAL, dout), jnp.float32),
    )(p, b2d)


def _slab_copy(src_ref, dst_ref, s):
    @pl.when(s < 8)
    def _():
        b = pl.multiple_of(s * SLAB_A, 8)
        pltpu.sync_copy(src_ref.at[pl.ds(b, SLAB_A)],
                        dst_ref.at[pl.ds(b, SLAB_A)])

    @pl.when(s >= 8)
    def _():
        b = pl.multiple_of(8 * SLAB_A + (s - 8) * SLAB_B, 8)
        pltpu.sync_copy(src_ref.at[pl.ds(b, SLAB_B)],
                        dst_ref.at[pl.ds(b, SLAB_B)])


def _seg_sum(y, src4, dst4, zeros, w, csz, nslab=2, stage=False):
    mesh = plsc.VectorSubcoreMesh(core_axis_name="c", subcore_axis_name="s")

    nbuf = 4
    chunks = EPT // csz
    spc = nslab // 2

    @functools.partial(
        pl.kernel,
        out_type=jax.ShapeDtypeStruct((nslab, NPAD, w), jnp.float32),
        mesh=mesh,
        scratch_types=(
            [pltpu.VMEM((chunks, 1, csz), jnp.int32),
             pltpu.VMEM((chunks, 1, csz), jnp.int32)]
            + [pltpu.VMEM((csz, w), jnp.float32)] * nbuf
            + [pltpu.VMEM_SHARED((NPAD, w), jnp.float32)]
            + ([pltpu.VMEM_SHARED((NPAD, w), jnp.float32)] if stage else [])
            + [pltpu.SemaphoreType.DMA] * (2 * nbuf)
        ),
        compiler_params=pltpu.CompilerParams(use_tc_tiling_on_sc=False),
    )
    def k(y_hbm, src_hbm, dst_hbm, z_hbm, out, *rest):
        if stage:
            (src_v, dst_v, r0, r1, r2, r3, acc_sh, y_sh,
             g0, g1, g2, g3, s0, s1, s2, s3) = rest
        else:
            (src_v, dst_v, r0, r1, r2, r3, acc_sh,
             g0, g1, g2, g3, s0, s1, s2, s3) = rest
            y_sh = None
        bufs = (r0, r1, r2, r3)
        gsem = (g0, g1, g2, g3)
        ssem = (s0, s1, s2, s3)
        c = lax.axis_index("c")
        s = lax.axis_index("s")
        pltpu.sync_copy(src_hbm.at[s], src_v)
        pltpu.sync_copy(dst_hbm.at[s], dst_v)

        def one_pass(sl):
            _slab_copy(z_hbm, acc_sh, s)
            if stage:
                _slab_copy(y_hbm.at[sl], y_sh, s)
            plsc.subcore_barrier()

            def ysrc(b):
                return y_sh if stage else y_hbm.at[sl]

            def gather_start(j, b):
                pltpu.async_copy(ysrc(b).at[src_v.at[j, 0]], bufs[b],
                                 gsem[b])

            def gather_wait(b):
                pltpu.make_async_copy(ysrc(b).at[src_v.at[0, 0]], bufs[b],
                                      gsem[b]).wait()

            def scatter_start(j, b):
                pltpu.async_copy(bufs[b], acc_sh.at[dst_v.at[j, 0]],
                                 ssem[b], add=True)

            def scatter_wait(b):
                pltpu.make_async_copy(bufs[b], acc_sh.at[dst_v.at[0, 0]],
                                      ssem[b]).wait()

            for b in range(nbuf):
                gather_start(b, b)

            def body(u, carry):
                j = 4 * u
                for b in range(nbuf):
                    gather_wait(b)
                    scatter_start(j + b, b)
                for b in range(nbuf):
                    @pl.when(j + nbuf + b < chunks)
                    def _(b=b):
                        scatter_wait(b)
                        gather_start(j + nbuf + b, b)
                return carry

            lax.fori_loop(0, chunks // 4, body, 0)
            for b in range(nbuf):
                scatter_wait(b)
            plsc.subcore_barrier()
            _slab_copy(acc_sh, out.at[sl], s)

        for q in range(spc):
            one_pass(spc * c + q if spc > 1 else c)

    return k(y, src4, dst4, zeros)


def _pad_w(m, r, c):
    return jnp.pad(m.astype(jnp.float32),
                   ((0, r - m.shape[0]), (0, c - m.shape[1])))


def kernel(features, edge_index, W1, b1, W2, b2, W3, b3):
    f32 = jnp.float32
    d1, d2, d3 = 256, 128, 64

    src = edge_index[0].astype(jnp.int32)
    dst = edge_index[1].astype(jnp.int32)
    pad = jnp.full((EPAD - E_REAL,), N_REAL, jnp.int32)
    srcp = jnp.concatenate([src, pad])
    dstp = jnp.concatenate([dst, pad])
    src80 = srcp.reshape(16, EPT // 80, 1, 80)
    dst80 = dstp.reshape(16, EPT // 80, 1, 80)
    src128 = srcp.reshape(16, EPT // 128, 1, 128)
    dst128 = dstp.reshape(16, EPT // 128, 1, 128)
    w1p = _pad_w(W1, d1, features.shape[1])
    w2p = _pad_w(W2, d2, d1)
    w3p = _pad_w(W3, d3, d2)
    b1p = jnp.pad(b1.astype(f32), (0, d1 - b1.shape[0])).reshape(1, d1)
    b2p = jnp.pad(b2.astype(f32), (0, d2 - b2.shape[0])).reshape(1, d2)
    b3p = jnp.pad(b3.astype(f32), (0, d3 - b3.shape[0])).reshape(1, d3)

    zeros64 = jnp.zeros((NPAD, 64), f32)
    y1 = _mm_first(features.astype(f32), w1p, 4)
    p = _seg_sum(y1, src80, dst80, zeros64, 64, 80, nslab=4, stage=True)
    y2 = _mm_fused(p, b1p, w2p, 2)
    p = _seg_sum(y2, src80, dst80, zeros64, 64, 80, nslab=2, stage=True)
    y3 = _mm_fused(p, b2p, w3p, 2)
    p = _seg_sum(y3, src128, dst128, jnp.zeros((NPAD, 32), f32),
                 32, 128, nslab=2, stage=True)
    return _final_act(p, b3p, W3.shape[0])


# speedup vs baseline: 1.1438x; 1.0001x over previous
import functools

import jax
import jax.numpy as jnp
from jax import lax
from jax.experimental import pallas as pl
from jax.experimental.pallas import tpu as pltpu
from jax.experimental.pallas import tpu_sc as plsc

N_REAL = 10000
NPAD = 10048
E_REAL = 160000
EPAD = 163840
EPT = EPAD // 16
SLAB_A = 632
SLAB_B = 624
BR = 1256


def _mm_first(x, wp, nslab):
    n, k = x.shape
    dout = wp.shape[0]
    w = dout // nslab

    def body(x_ref, w_ref, o_ref):
        i = pl.program_id(0)
        y = lax.dot_general(x_ref[...], w_ref[...], (((1,), (1,)), ((), ())),
                            preferred_element_type=jnp.float32)
        row = i * BR + lax.broadcasted_iota(jnp.int32, (BR, dout), 0)
        y = jnp.where(row < n, y, 0.0)
        for q in range(nslab):
            o_ref[q] = y[:, q * w:(q + 1) * w]

    return pl.pallas_call(
        body,
        grid=(NPAD // BR,),
        in_specs=[pl.BlockSpec((BR, k), lambda i: (i, 0)),
                  pl.BlockSpec((dout, k), lambda i: (0, 0))],
        out_specs=pl.BlockSpec((nslab, BR, w), lambda i: (0, i, 0)),
        out_shape=jax.ShapeDtypeStruct((nslab, NPAD, w), jnp.float32),
    )(x, wp)


def _mm_fused(p, b2d, wp, nslab):
    inslab, n, win = p.shape
    din = inslab * win
    dout = wp.shape[0]
    w = dout // nslab

    def body(p_ref, b_ref, w_ref, o_ref):
        i = pl.program_id(0)
        h = jnp.concatenate([p_ref[q] for q in range(inslab)], axis=1)
        h = jnp.maximum(h + b_ref[...], 0.0)
        y = lax.dot_general(h, w_ref[...], (((1,), (1,)), ((), ())),
                            preferred_element_type=jnp.float32)
        row = i * BR + lax.broadcasted_iota(jnp.int32, (BR, dout), 0)
        y = jnp.where(row < N_REAL, y, 0.0)
        for q in range(nslab):
            o_ref[q] = y[:, q * w:(q + 1) * w]

    return pl.pallas_call(
        body,
        grid=(n // BR,),
        in_specs=[pl.BlockSpec((inslab, BR, win), lambda i: (0, i, 0)),
                  pl.BlockSpec((1, din), lambda i: (0, 0)),
                  pl.BlockSpec((dout, din), lambda i: (0, 0))],
        out_specs=pl.BlockSpec((nslab, BR, w), lambda i: (0, i, 0)),
        out_shape=jax.ShapeDtypeStruct((nslab, n, w), jnp.float32),
    )(p, b2d, wp)


def _final_act(p, b2d, dout):
    inslab, n, win = p.shape
    din = inslab * win

    def body(p_ref, b_ref, o_ref):
        h = jnp.concatenate([p_ref[q] for q in range(inslab)], axis=1)
        h = jnp.maximum(h + b_ref[...], 0.0)
        o_ref[...] = h[:, :dout]

    return pl.pallas_call(
        body,
        grid=(n // BR,),
        in_specs=[pl.BlockSpec((inslab, BR, win), lambda i: (0, i, 0)),
                  pl.BlockSpec((1, din), lambda i: (0, 0))],
        out_specs=pl.BlockSpec((BR, dout), lambda i: (i, 0)),
        out_shape=jax.ShapeDtypeStruct((N_REAL, dout), jnp.float32),
    )(p, b2d)


def _slab_copy(src_ref, dst_ref, s):
    @pl.when(s < 8)
    def _():
        b = pl.multiple_of(s * SLAB_A, 8)
        pltpu.sync_copy(src_ref.at[pl.ds(b, SLAB_A)],
                        dst_ref.at[pl.ds(b, SLAB_A)])

    @pl.when(s >= 8)
    def _():
        b = pl.multiple_of(8 * SLAB_A + (s - 8) * SLAB_B, 8)
        pltpu.sync_copy(src_ref.at[pl.ds(b, SLAB_B)],
                        dst_ref.at[pl.ds(b, SLAB_B)])


def _seg_sum(y, src4, dst4, zeros, w, csz, nslab=2, stage=False):
    mesh = plsc.VectorSubcoreMesh(core_axis_name="c", subcore_axis_name="s")

    nbuf = 4
    chunks = EPT // csz
    spc = nslab // 2

    @functools.partial(
        pl.kernel,
        out_type=jax.ShapeDtypeStruct((nslab, NPAD, w), jnp.float32),
        mesh=mesh,
        scratch_types=(
            [pltpu.VMEM((chunks, 1, csz), jnp.int32),
             pltpu.VMEM((chunks, 1, csz), jnp.int32)]
            + [pltpu.VMEM((csz, w), jnp.float32)] * nbuf
            + [pltpu.VMEM_SHARED((NPAD, w), jnp.float32)]
            + ([pltpu.VMEM_SHARED((NPAD, w), jnp.float32)] if stage else [])
            + [pltpu.SemaphoreType.DMA] * (2 * nbuf)
        ),
        compiler_params=pltpu.CompilerParams(use_tc_tiling_on_sc=False),
    )
    def k(y_hbm, src_hbm, dst_hbm, z_hbm, out, *rest):
        if stage:
            (src_v, dst_v, r0, r1, r2, r3, acc_sh, y_sh,
             g0, g1, g2, g3, s0, s1, s2, s3) = rest
        else:
            (src_v, dst_v, r0, r1, r2, r3, acc_sh,
             g0, g1, g2, g3, s0, s1, s2, s3) = rest
            y_sh = None
        bufs = (r0, r1, r2, r3)
        gsem = (g0, g1, g2, g3)
        ssem = (s0, s1, s2, s3)
        c = lax.axis_index("c")
        s = lax.axis_index("s")
        pltpu.sync_copy(src_hbm.at[s], src_v)
        pltpu.sync_copy(dst_hbm.at[s], dst_v)

        def one_pass(sl):
            _slab_copy(z_hbm, acc_sh, s)
            if stage:
                _slab_copy(y_hbm.at[sl], y_sh, s)
            plsc.subcore_barrier()

            def ysrc(b):
                return y_sh if stage else y_hbm.at[sl]

            def gather_start(j, b):
                pltpu.async_copy(ysrc(b).at[src_v.at[j, 0]], bufs[b],
                                 gsem[b])

            def gather_wait(b):
                pltpu.make_async_copy(ysrc(b).at[src_v.at[0, 0]], bufs[b],
                                      gsem[b]).wait()

            def scatter_start(j, b):
                pltpu.async_copy(bufs[b], acc_sh.at[dst_v.at[j, 0]],
                                 ssem[b], add=True)

            def scatter_wait(b):
                pltpu.make_async_copy(bufs[b], acc_sh.at[dst_v.at[0, 0]],
                                      ssem[b]).wait()

            for b in range(nbuf):
                gather_start(b, b)

            def body(u, carry):
                j = 4 * u
                for b in range(nbuf):
                    gather_wait(b)
                    scatter_start(j + b, b)
                for b in range(nbuf):
                    @pl.when(j + nbuf + b < chunks)
                    def _(b=b):
                        scatter_wait(b)
                        gather_start(j + nbuf + b, b)
                return carry

            lax.fori_loop(0, chunks // 4, body, 0)
            for b in range(nbuf):
                scatter_wait(b)
            plsc.subcore_barrier()
            _slab_copy(acc_sh, out.at[sl], s)

        for q in range(spc):
            one_pass(spc * c + q if spc > 1 else c)

    return k(y, src4, dst4, zeros)


def _pad_w(m, r, c):
    return jnp.pad(m.astype(jnp.float32),
                   ((0, r - m.shape[0]), (0, c - m.shape[1])))


def kernel(features, edge_index, W1, b1, W2, b2, W3, b3):
    f32 = jnp.float32
    d1, d2, d3 = 256, 128, 64

    src = edge_index[0].astype(jnp.int32)
    dst = edge_index[1].astype(jnp.int32)
    pad = jnp.full((EPAD - E_REAL,), N_REAL, jnp.int32)
    src80 = jnp.concatenate([src, pad]).reshape(16, EPT // 80, 1, 80)
    dst80 = jnp.concatenate([dst, pad]).reshape(16, EPT // 80, 1, 80)
    w1p = _pad_w(W1, d1, features.shape[1])
    w2p = _pad_w(W2, d2, d1)
    w3p = _pad_w(W3, d3, d2)
    b1p = jnp.pad(b1.astype(f32), (0, d1 - b1.shape[0])).reshape(1, d1)
    b2p = jnp.pad(b2.astype(f32), (0, d2 - b2.shape[0])).reshape(1, d2)
    b3p = jnp.pad(b3.astype(f32), (0, d3 - b3.shape[0])).reshape(1, d3)

    zeros64 = jnp.zeros((NPAD, 64), f32)
    y1 = _mm_first(features.astype(f32), w1p, 4)
    p = _seg_sum(y1, src80, dst80, zeros64, 64, 80, nslab=4, stage=True)
    y2 = _mm_fused(p, b1p, w2p, 2)
    p = _seg_sum(y2, src80, dst80, zeros64, 64, 80, nslab=2, stage=True)
    y3 = _mm_fused(p, b2p, w3p, 2)
    p = _seg_sum(y3, src80, dst80, jnp.zeros((NPAD, 32), f32),
                 32, 80, nslab=2, stage=True)
    return _final_act(p, b3p, W3.shape[0])
